# baseline (speedup 1.0000x reference)
import functools

import jax
import jax.numpy as jnp
from jax import lax
from jax.experimental import pallas as pl
from jax.experimental.pallas import tpu as pltpu
from jax.experimental.pallas import tpu_sc as plsc

_N = 10000
_E = 320000
_D = 128
_NC = 2
_NS = 16
_HD = _D // _NC

_C = 128
_GPC = _C // 16
_W = 6
_NCHUNK = 168
_NMACRO = _NCHUNK // _W
_EPT = _NCHUNK * _C
_EPAD = _EPT * _NS
_NPAD = 10240
_NPT = _NPAD // _NS

_NGB = 3
_NSB = 2
_NSL = 3


def _spmm_pass(src_table, acc, cols_hbm, rows_hbm, w_hbm, cslab, rslab,
               wslab, gbuf, sbuf, isems, gsems, ssems, s):
    tbase = s * _NCHUNK

    def fire_slab(m, sl):
        pltpu.async_copy(cols_hbm.at[pl.ds(tbase + m * _W, _W)],
                         cslab.at[sl], isems.at[sl])
        pltpu.async_copy(rows_hbm.at[pl.ds(tbase + m * _W, _W)],
                         rslab.at[sl], isems.at[sl])
        pltpu.async_copy(w_hbm.at[pl.ds((tbase + m * _W) * _GPC, _W * _GPC)],
                         wslab.at[sl], isems.at[sl])

    def drain_slab(sl):
        pltpu.make_async_copy(cols_hbm.at[pl.ds(0, _W)], cslab.at[sl],
                              isems.at[sl]).wait()
        pltpu.make_async_copy(rows_hbm.at[pl.ds(0, _W)], rslab.at[sl],
                              isems.at[sl]).wait()
        pltpu.make_async_copy(w_hbm.at[pl.ds(0, _W * _GPC)], wslab.at[sl],
                              isems.at[sl]).wait()

    def fire_gather(sl, jj, gb):
        pltpu.async_copy(src_table.at[cslab.at[sl, jj]],
                         gbuf.at[pl.ds(gb * _C, _C)], gsems.at[gb])

    def drain_gather(gb):
        pltpu.make_async_copy(src_table.at[cslab.at[0, 0]],
                              gbuf.at[pl.ds(gb * _C, _C)],
                              gsems.at[gb]).wait()

    def fire_scatter(sl, jj, sb):
        pltpu.async_copy(sbuf.at[pl.ds(sb * _C, _C)],
                         acc.at[rslab.at[sl, jj]], ssems.at[sb], add=True)

    def drain_scatter(sb):
        pltpu.make_async_copy(sbuf.at[pl.ds(sb * _C, _C)],
                              acc.at[rslab.at[0, 0]],
                              ssems.at[sb]).wait()

    fire_slab(0, 0)
    fire_slab(1, 1)
    drain_slab(0)
    fire_gather(0, 0, 0)
    fire_gather(0, 1, 1)

    def pipeline_macro(m, _):
        sl = lax.rem(m, _NSL)
        sln = lax.rem(m + 1, _NSL)
        for j in range(_W):
            k = m * _W + j
            gb = j % _NGB
            sb = j % _NSB

            if j == 2:
                @pl.when(m + 2 < _NMACRO)
                def _():
                    fire_slab(m + 2, lax.rem(m + 2, _NSL))

            if j == 4:
                @pl.when(m + 1 < _NMACRO)
                def _():
                    drain_slab(sln)

            jj2 = (j + 2) % _W
            gb2 = (j + 2) % _NGB
            sl2 = sln if j >= _W - 2 else sl

            @pl.when(k + 2 < _NCHUNK)
            def _(jj2=jj2, gb2=gb2, sl2=sl2):
                fire_gather(sl2, jj2, gb2)

            drain_gather(gb)

            @pl.when(k >= _NSB)
            def _(sb=sb):
                drain_scatter(sb)

            @plsc.parallel_loop(0, _GPC, unroll=2)
            def _(g, gb=gb, sb=sb, sl=sl, j=j):
                w16 = wslab[sl, j * _GPC + g]
                for e in range(16):
                    ws = jnp.full((16,), w16[e], jnp.float32)
                    rs = g * 16 + e
                    for f in range(_HD // 16):
                        sbuf[sb * _C + rs, pl.ds(f * 16, 16)] = (
                            gbuf[gb * _C + rs, pl.ds(f * 16, 16)] * ws)

            fire_scatter(sl, j, sb)
        return 0

    lax.fori_loop(0, _NMACRO, pipeline_macro, 0)

    drain_scatter(0)
    drain_scatter(1)


def _sg_kernel(x3, rows_hbm, cols_hbm, w_hbm, out, acc1, acc2, cslab, rslab,
               wslab, gbuf, sbuf, isems, gsems, ssems):
    c = lax.axis_index("c")
    s = lax.axis_index("s")

    def zfill(r, _):
        for fb in range(_HD // 16):
            sbuf[r, pl.ds(fb * 16, 16)] = jnp.zeros((16,), jnp.float32)
        return 0

    lax.fori_loop(0, _NSB * _C, zfill, 0)
    for off, nrow in ((0, 256), (256, 256), (512, 128)):
        pltpu.sync_copy(sbuf.at[pl.ds(0, nrow)],
                        acc1.at[pl.ds(s * _NPT + off, nrow)])
        pltpu.sync_copy(sbuf.at[pl.ds(0, nrow)],
                        acc2.at[pl.ds(s * _NPT + off, nrow)])
    plsc.subcore_barrier()

    _spmm_pass(x3.at[c], acc1, cols_hbm, rows_hbm, w_hbm, cslab, rslab,
               wslab, gbuf, sbuf, isems, gsems, ssems, s)
    pltpu.sync_copy(acc1.at[pl.ds(s * _NPT, _NPT)],
                    out.at[c, pl.ds(s * _NPT, _NPT)])
    plsc.subcore_barrier()

    _spmm_pass(out.at[c], acc2, cols_hbm, rows_hbm, w_hbm, cslab, rslab,
               wslab, gbuf, sbuf, isems, gsems, ssems, s)
    plsc.subcore_barrier()

    pltpu.sync_copy(acc2.at[pl.ds(s * _NPT, _NPT)],
                    out.at[c, pl.ds(s * _NPT, _NPT)])


@functools.cache
def _sg_call():
    return pl.kernel(
        _sg_kernel,
        out_type=jax.ShapeDtypeStruct((_NC, _NPAD, _HD), jnp.float32),
        mesh=plsc.VectorSubcoreMesh(core_axis_name="c", subcore_axis_name="s",
                                    num_cores=_NC, num_subcores=_NS),
        scratch_types=[
            pltpu.VMEM_SHARED((_NPAD, _HD), jnp.float32),
            pltpu.VMEM_SHARED((_NPAD, _HD), jnp.float32),
            pltpu.VMEM((_NSL, _W, 128), jnp.int32),
            pltpu.VMEM((_NSL, _W, 128), jnp.int32),
            pltpu.VMEM((_NSL, _W * _GPC, 16), jnp.float32),
            pltpu.VMEM((_NGB * _C, _HD), jnp.float32),
            pltpu.VMEM((_NSB * _C, _HD), jnp.float32),
            pltpu.SemaphoreType.DMA((_NSL,)),
            pltpu.SemaphoreType.DMA((_NGB,)),
            pltpu.SemaphoreType.DMA((_NSB,)),
        ],
        compiler_params=pltpu.CompilerParams(use_tc_tiling_on_sc=False),
    )


def kernel(x, edge_index, edge_weight):
    x3 = jnp.transpose(x.reshape(_N, _NC, _HD), (1, 0, 2))
    rows = edge_index[0].astype(jnp.int32)
    cols = edge_index[1].astype(jnp.int32)
    w = edge_weight.astype(jnp.float32)
    pad = _EPAD - _E
    rows_p = jnp.concatenate([rows, jnp.zeros((pad,), jnp.int32)])
    cols_p = jnp.concatenate([cols, jnp.zeros((pad,), jnp.int32)])
    w_p = jnp.concatenate([w, jnp.zeros((pad,), jnp.float32)])
    out = _sg_call()(x3,
                     rows_p.reshape(_EPAD // 128, 128),
                     cols_p.reshape(_EPAD // 128, 128),
                     w_p.reshape(_EPAD // 16, 16))
    return jnp.transpose(out[:, :_N], (1, 0, 2)).reshape(_N, _D)


# speedup vs baseline: 3.5479x; 3.5479x over previous
import functools

import jax
import jax.numpy as jnp
from jax import lax
from jax.experimental import pallas as pl
from jax.experimental.pallas import tpu as pltpu
from jax.experimental.pallas import tpu_sc as plsc

_N = 10000
_E = 320000
_D = 128
_NC = 2
_NS = 16
_HD = _D // _NC

_C = 128
_GPC = _C // 16
_W = 6
_NCHUNK = 168
_NMACRO = _NCHUNK // _W
_EPT = _NCHUNK * _C
_EPAD = _EPT * _NS
_NPAD = 10240
_NPT = _NPAD // _NS

_NGB = 3
_NSB = 2
_NSL = 3


def _spmm_pass(src_table, acc, cols_hbm, rows_hbm, w_hbm, cslab, rslab,
               wslab, gbuf, sbuf, isems, gsems, ssems, s):
    tbase = s * _NCHUNK

    def fire_slab(m, sl):
        pltpu.async_copy(cols_hbm.at[pl.ds(tbase + m * _W, _W)],
                         cslab.at[sl], isems.at[sl])
        pltpu.async_copy(rows_hbm.at[pl.ds(tbase + m * _W, _W)],
                         rslab.at[sl], isems.at[sl])
        pltpu.async_copy(w_hbm.at[pl.ds((tbase + m * _W) * _GPC, _W * _GPC)],
                         wslab.at[sl], isems.at[sl])

    def drain_slab(sl):
        pltpu.make_async_copy(cols_hbm.at[pl.ds(0, _W)], cslab.at[sl],
                              isems.at[sl]).wait()
        pltpu.make_async_copy(rows_hbm.at[pl.ds(0, _W)], rslab.at[sl],
                              isems.at[sl]).wait()
        pltpu.make_async_copy(w_hbm.at[pl.ds(0, _W * _GPC)], wslab.at[sl],
                              isems.at[sl]).wait()

    def fire_gather(sl, jj, gb):
        pltpu.async_copy(src_table.at[cslab.at[sl, jj]],
                         gbuf.at[pl.ds(gb * _C, _C)], gsems.at[gb])

    def drain_gather(gb):
        pltpu.make_async_copy(src_table.at[cslab.at[0, 0]],
                              gbuf.at[pl.ds(gb * _C, _C)],
                              gsems.at[gb]).wait()

    def fire_scatter(sl, jj, sb):
        pltpu.async_copy(sbuf.at[pl.ds(sb * _C, _C)],
                         acc.at[rslab.at[sl, jj]], ssems.at[sb], add=True)

    def drain_scatter(sb):
        pltpu.make_async_copy(sbuf.at[pl.ds(sb * _C, _C)],
                              acc.at[rslab.at[0, 0]],
                              ssems.at[sb]).wait()

    fire_slab(0, 0)
    fire_slab(1, 1)
    drain_slab(0)
    fire_gather(0, 0, 0)
    fire_gather(0, 1, 1)

    def pipeline_macro(m, _):
        sl = lax.rem(m, _NSL)
        sln = lax.rem(m + 1, _NSL)
        for j in range(_W):
            k = m * _W + j
            gb = j % _NGB
            sb = j % _NSB

            if j == 2:
                @pl.when(m + 2 < _NMACRO)
                def _():
                    fire_slab(m + 2, lax.rem(m + 2, _NSL))

            if j == 4:
                @pl.when(m + 1 < _NMACRO)
                def _():
                    drain_slab(sln)

            jj2 = (j + 2) % _W
            gb2 = (j + 2) % _NGB
            sl2 = sln if j >= _W - 2 else sl

            @pl.when(k + 2 < _NCHUNK)
            def _(jj2=jj2, gb2=gb2, sl2=sl2):
                fire_gather(sl2, jj2, gb2)

            drain_gather(gb)

            @pl.when(k >= _NSB)
            def _(sb=sb):
                drain_scatter(sb)

            @plsc.parallel_loop(0, _GPC, unroll=2)
            def _(g, gb=gb, sb=sb, sl=sl, j=j):
                w16 = wslab[sl, j * _GPC + g]
                for e in range(16):
                    ws = jnp.full((16,), w16[e], jnp.float32)
                    rs = g * 16 + e
                    for f in range(_HD // 16):
                        sbuf[sb * _C + rs, pl.ds(f * 16, 16)] = (
                            gbuf[gb * _C + rs, pl.ds(f * 16, 16)] * ws)

            fire_scatter(sl, j, sb)
        return 0

    lax.fori_loop(0, _NMACRO, pipeline_macro, 0)

    drain_scatter(0)
    drain_scatter(1)


def _sg_kernel(x3, rows_hbm, cols_hbm, w_hbm, out, taba, tabb, cslab, rslab,
               wslab, gbuf, sbuf, isems, gsems, ssems):
    c = lax.axis_index("c")
    s = lax.axis_index("s")

    def zero_sbuf():
        def zfill(r, _):
            for fb in range(_HD // 16):
                sbuf[r, pl.ds(fb * 16, 16)] = jnp.zeros((16,), jnp.float32)
            return 0

        lax.fori_loop(0, _NSB * _C, zfill, 0)

    def zero_stripe(tab):
        for off, nrow in ((0, 256), (256, 256), (512, 128)):
            pltpu.sync_copy(sbuf.at[pl.ds(0, nrow)],
                            tab.at[pl.ds(s * _NPT + off, nrow)])

    pltpu.sync_copy(x3.at[c, pl.ds(s * _NPT, _NPT)],
                    taba.at[pl.ds(s * _NPT, _NPT)])
    zero_sbuf()
    zero_stripe(tabb)
    plsc.subcore_barrier()

    _spmm_pass(taba, tabb, cols_hbm, rows_hbm, w_hbm, cslab, rslab,
               wslab, gbuf, sbuf, isems, gsems, ssems, s)
    plsc.subcore_barrier()

    zero_sbuf()
    zero_stripe(taba)
    plsc.subcore_barrier()

    _spmm_pass(tabb, taba, cols_hbm, rows_hbm, w_hbm, cslab, rslab,
               wslab, gbuf, sbuf, isems, gsems, ssems, s)
    plsc.subcore_barrier()

    pltpu.sync_copy(taba.at[pl.ds(s * _NPT, _NPT)],
                    out.at[c, pl.ds(s * _NPT, _NPT)])


@functools.cache
def _sg_call():
    return pl.kernel(
        _sg_kernel,
        out_type=jax.ShapeDtypeStruct((_NC, _NPAD, _HD), jnp.float32),
        mesh=plsc.VectorSubcoreMesh(core_axis_name="c", subcore_axis_name="s",
                                    num_cores=_NC, num_subcores=_NS),
        scratch_types=[
            pltpu.VMEM_SHARED((_NPAD, _HD), jnp.float32),
            pltpu.VMEM_SHARED((_NPAD, _HD), jnp.float32),
            pltpu.VMEM((_NSL, _W, 128), jnp.int32),
            pltpu.VMEM((_NSL, _W, 128), jnp.int32),
            pltpu.VMEM((_NSL, _W * _GPC, 16), jnp.float32),
            pltpu.VMEM((_NGB * _C, _HD), jnp.float32),
            pltpu.VMEM((_NSB * _C, _HD), jnp.float32),
            pltpu.SemaphoreType.DMA((_NSL,)),
            pltpu.SemaphoreType.DMA((_NGB,)),
            pltpu.SemaphoreType.DMA((_NSB,)),
        ],
        compiler_params=pltpu.CompilerParams(use_tc_tiling_on_sc=False),
    )


def kernel(x, edge_index, edge_weight):
    x3 = jnp.transpose(x.reshape(_N, _NC, _HD), (1, 0, 2))
    x3 = jnp.pad(x3, ((0, 0), (0, _NPAD - _N), (0, 0)))
    rows = edge_index[0].astype(jnp.int32)
    cols = edge_index[1].astype(jnp.int32)
    w = edge_weight.astype(jnp.float32)
    pad = _EPAD - _E
    rows_p = jnp.concatenate([rows, jnp.zeros((pad,), jnp.int32)])
    cols_p = jnp.concatenate([cols, jnp.zeros((pad,), jnp.int32)])
    w_p = jnp.concatenate([w, jnp.zeros((pad,), jnp.float32)])
    out = _sg_call()(x3,
                     rows_p.reshape(_EPAD // 128, 128),
                     cols_p.reshape(_EPAD // 128, 128),
                     w_p.reshape(_EPAD // 16, 16))
    return jnp.transpose(out[:, :_N], (1, 0, 2)).reshape(_N, _D)


# speedup vs baseline: 4.0704x; 1.1473x over previous
import functools

import jax
import jax.numpy as jnp
from jax import lax
from jax.experimental import pallas as pl
from jax.experimental.pallas import tpu as pltpu
from jax.experimental.pallas import tpu_sc as plsc

_N = 10000
_E = 320000
_D = 128
_NC = 2
_NS = 16
_HD = _D // _NC

_C = 128
_GPC = _C // 16
_W = 6
_NCHUNK = 168
_NMACRO = _NCHUNK // _W
_EPT = _NCHUNK * _C
_EPAD = _EPT * _NS
_NPAD = 10240
_NPT = _NPAD // _NS

_NGB = 3
_NSB = 2
_NSL = 3


def _spmm_pass(src_table, acc, cols_hbm, rows_hbm, w_hbm, cslab, rslab,
               wslab, gbuf, sbuf, isems, gsems, ssems, s):
    tbase = s * _NCHUNK

    def fire_slab(m, sl):
        pltpu.async_copy(cols_hbm.at[pl.ds(tbase + m * _W, _W)],
                         cslab.at[sl], isems.at[sl])
        pltpu.async_copy(rows_hbm.at[pl.ds(tbase + m * _W, _W)],
                         rslab.at[sl], isems.at[sl])
        pltpu.async_copy(w_hbm.at[pl.ds((tbase + m * _W) * _GPC, _W * _GPC)],
                         wslab.at[sl], isems.at[sl])

    def drain_slab(sl):
        pltpu.make_async_copy(cols_hbm.at[pl.ds(0, _W)], cslab.at[sl],
                              isems.at[sl]).wait()
        pltpu.make_async_copy(rows_hbm.at[pl.ds(0, _W)], rslab.at[sl],
                              isems.at[sl]).wait()
        pltpu.make_async_copy(w_hbm.at[pl.ds(0, _W * _GPC)], wslab.at[sl],
                              isems.at[sl]).wait()

    def fire_gather(sl, jj, gb):
        pltpu.async_copy(src_table.at[cslab.at[sl, jj]],
                         gbuf.at[pl.ds(gb * _C, _C)], gsems.at[gb])

    def drain_gather(gb):
        pltpu.make_async_copy(src_table.at[cslab.at[0, 0]],
                              gbuf.at[pl.ds(gb * _C, _C)],
                              gsems.at[gb]).wait()

    def fire_scatter(sl, jj, sb):
        pltpu.async_copy(sbuf.at[pl.ds(sb * _C, _C)],
                         acc.at[rslab.at[sl, jj]], ssems.at[sb], add=True)

    def drain_scatter(sb):
        pltpu.make_async_copy(sbuf.at[pl.ds(sb * _C, _C)],
                              acc.at[rslab.at[0, 0]],
                              ssems.at[sb]).wait()

    fire_slab(0, 0)
    fire_slab(1, 1)
    drain_slab(0)
    fire_gather(0, 0, 0)
    fire_gather(0, 1, 1)

    def pipeline_macro(m, _):
        sl = lax.rem(m, _NSL)
        sln = lax.rem(m + 1, _NSL)
        for j in range(_W):
            k = m * _W + j
            gb = j % _NGB
            sb = j % _NSB

            if j == 2:
                @pl.when(m + 2 < _NMACRO)
                def _():
                    fire_slab(m + 2, lax.rem(m + 2, _NSL))

            if j == 4:
                @pl.when(m + 1 < _NMACRO)
                def _():
                    drain_slab(sln)

            jj2 = (j + 2) % _W
            gb2 = (j + 2) % _NGB
            sl2 = sln if j >= _W - 2 else sl

            @pl.when(k + 2 < _NCHUNK)
            def _(jj2=jj2, gb2=gb2, sl2=sl2):
                fire_gather(sl2, jj2, gb2)

            drain_gather(gb)

            @pl.when(k >= _NSB)
            def _(sb=sb):
                drain_scatter(sb)

            @plsc.parallel_loop(0, 1, unroll=1)
            def _(g, gb=gb, sb=sb, sl=sl, j=j):
                w16 = wslab[sl, j * _GPC + g]
                for e in range(16):
                    ws = jnp.full((16,), w16[e], jnp.float32)
                    rs = g * 16 + e
                    for f in range(_HD // 16):
                        sbuf[sb * _C + rs, pl.ds(f * 16, 16)] = (
                            gbuf[gb * _C + rs, pl.ds(f * 16, 16)] * ws)

            fire_scatter(sl, j, sb)
        return 0

    lax.fori_loop(0, _NMACRO, pipeline_macro, 0)

    drain_scatter(0)
    drain_scatter(1)


def _sg_kernel(x3, rows_hbm, cols_hbm, w_hbm, out, taba, tabb, cslab, rslab,
               wslab, gbuf, sbuf, isems, gsems, ssems):
    c = lax.axis_index("c")
    s = lax.axis_index("s")

    def zero_sbuf():
        def zfill(r, _):
            for fb in range(_HD // 16):
                sbuf[r, pl.ds(fb * 16, 16)] = jnp.zeros((16,), jnp.float32)
            return 0

        lax.fori_loop(0, _NSB * _C, zfill, 0)

    def zero_stripe(tab):
        for off, nrow in ((0, 256), (256, 256), (512, 128)):
            pltpu.sync_copy(sbuf.at[pl.ds(0, nrow)],
                            tab.at[pl.ds(s * _NPT + off, nrow)])

    pltpu.sync_copy(x3.at[c, pl.ds(s * _NPT, _NPT)],
                    taba.at[pl.ds(s * _NPT, _NPT)])
    zero_sbuf()
    zero_stripe(tabb)
    plsc.subcore_barrier()

    _spmm_pass(taba, tabb, cols_hbm, rows_hbm, w_hbm, cslab, rslab,
               wslab, gbuf, sbuf, isems, gsems, ssems, s)
    plsc.subcore_barrier()

    zero_sbuf()
    zero_stripe(taba)
    plsc.subcore_barrier()

    _spmm_pass(tabb, taba, cols_hbm, rows_hbm, w_hbm, cslab, rslab,
               wslab, gbuf, sbuf, isems, gsems, ssems, s)
    plsc.subcore_barrier()

    pltpu.sync_copy(taba.at[pl.ds(s * _NPT, _NPT)],
                    out.at[c, pl.ds(s * _NPT, _NPT)])


@functools.cache
def _sg_call():
    return pl.kernel(
        _sg_kernel,
        out_type=jax.ShapeDtypeStruct((_NC, _NPAD, _HD), jnp.float32),
        mesh=plsc.VectorSubcoreMesh(core_axis_name="c", subcore_axis_name="s",
                                    num_cores=_NC, num_subcores=_NS),
        scratch_types=[
            pltpu.VMEM_SHARED((_NPAD, _HD), jnp.float32),
            pltpu.VMEM_SHARED((_NPAD, _HD), jnp.float32),
            pltpu.VMEM((_NSL, _W, 128), jnp.int32),
            pltpu.VMEM((_NSL, _W, 128), jnp.int32),
            pltpu.VMEM((_NSL, _W * _GPC, 16), jnp.float32),
            pltpu.VMEM((_NGB * _C, _HD), jnp.float32),
            pltpu.VMEM((_NSB * _C, _HD), jnp.float32),
            pltpu.SemaphoreType.DMA((_NSL,)),
            pltpu.SemaphoreType.DMA((_NGB,)),
            pltpu.SemaphoreType.DMA((_NSB,)),
        ],
        compiler_params=pltpu.CompilerParams(use_tc_tiling_on_sc=False),
    )


def kernel(x, edge_index, edge_weight):
    x3 = jnp.transpose(x.reshape(_N, _NC, _HD), (1, 0, 2))
    x3 = jnp.pad(x3, ((0, 0), (0, _NPAD - _N), (0, 0)))
    rows = edge_index[0].astype(jnp.int32)
    cols = edge_index[1].astype(jnp.int32)
    w = edge_weight.astype(jnp.float32)
    pad = _EPAD - _E
    rows_p = jnp.concatenate([rows, jnp.zeros((pad,), jnp.int32)])
    cols_p = jnp.concatenate([cols, jnp.zeros((pad,), jnp.int32)])
    w_p = jnp.concatenate([w, jnp.zeros((pad,), jnp.float32)])
    out = _sg_call()(x3,
                     rows_p.reshape(_EPAD // 128, 128),
                     cols_p.reshape(_EPAD // 128, 128),
                     w_p.reshape(_EPAD // 16, 16))
    return jnp.transpose(out[:, :_N], (1, 0, 2)).reshape(_N, _D)
